# baseline (speedup 1.0000x reference)
import jax
import jax.numpy as jnp
from jax.experimental import pallas as pl

_FEAT = 128
_Q = 100000
_B = 256
_SCALE = 30.0
_MARGIN = 0.35
_W = 2048


def _norm_rows(x):
    n = jnp.sqrt(jnp.sum(x * x, axis=1, keepdims=True))
    return x / jnp.maximum(n, 1e-12)


def _body(np_ref, vg_ref, vp_ref, ng_ref, vq_ref, nq_ref,
          o1_ref, o2_ref, nvq_ref, nnq_ref):
    j = pl.program_id(0)
    npn = _norm_rows(np_ref[...])
    vpn = _norm_rows(vp_ref[...])
    vq = vq_ref[...]
    nq = nq_ref[...]
    c1 = jnp.clip(jnp.dot(npn, vq, preferred_element_type=jnp.float32), -1.0, 1.0)
    c2 = jnp.clip(jnp.dot(vpn, nq, preferred_element_type=jnp.float32), -1.0, 1.0)
    o1_ref[...] = _SCALE * c1
    o2_ref[...] = _SCALE * c2
    nvq_ref[...] = vq
    nnq_ref[...] = nq

    @pl.when(j == 0)
    def _first_block():
        vgn = _norm_rows(vg_ref[...])
        ngn = _norm_rows(ng_ref[...])
        dn = (((1,), (1,)), ((), ()))
        g1 = jnp.clip(jax.lax.dot_general(npn, vgn, dn,
                                          preferred_element_type=jnp.float32),
                      -1.0, 1.0)
        g2 = jnp.clip(jax.lax.dot_general(vpn, ngn, dn,
                                          preferred_element_type=jnp.float32),
                      -1.0, 1.0)
        r = jax.lax.broadcasted_iota(jnp.int32, (_B, _B), 0)
        c = jax.lax.broadcasted_iota(jnp.int32, (_B, _B), 1)
        m = jnp.where(r == c, jnp.float32(_MARGIN), jnp.float32(0.0))
        o1_ref[:, :_B] = _SCALE * (g1 - m)
        o2_ref[:, :_B] = _SCALE * (g2 - m)
        nvq_ref[:, :_B] = vgn.T
        nnq_ref[:, :_B] = ngn.T


def kernel(nir_p, vis_g, vis_p, nir_g, cur_ids, vis_queue, nir_queue):
    nb = pl.cdiv(_Q, _W)
    full = pl.BlockSpec((_B, _FEAT), lambda j: (0, 0))
    colq = pl.BlockSpec((_FEAT, _W), lambda j: (0, j))
    colo = pl.BlockSpec((_B, _W), lambda j: (0, j))
    o1, o2, nvq, nnq = pl.pallas_call(
        _body,
        grid=(nb,),
        in_specs=[full, full, full, full, colq, colq],
        out_specs=(colo, colo, colq, colq),
        out_shape=(
            jax.ShapeDtypeStruct((_B, _Q), jnp.float32),
            jax.ShapeDtypeStruct((_B, _Q), jnp.float32),
            jax.ShapeDtypeStruct((_FEAT, _Q), jnp.float32),
            jax.ShapeDtypeStruct((_FEAT, _Q), jnp.float32),
        ),
    )(nir_p, vis_g, vis_p, nir_g, vis_queue, nir_queue)
    label = jnp.arange(_B, dtype=jnp.int32)
    return (o1, o2, label, nvq, nnq)
